# Initial kernel scaffold; baseline (speedup 1.0000x reference)
import jax
import jax.numpy as jnp
from jax.experimental import pallas as pl


def kernel(x, router_logits_a, router_logits_b, Wa, Wb):
    raise NotImplementedError("write your pallas kernel here")


# speedup vs baseline: 11.2927x; 11.2927x over previous
import functools

import jax
import jax.numpy as jnp
from jax.experimental import pallas as pl
from jax.experimental.pallas import tpu as pltpu

IN_FEATURES = 2048
OUT_FEATURES = 2048
R = 8
LORA_ALPHA = 16
NUM_EXPERTS = 8
TOP_K = 2
SCALING = LORA_ALPHA / R
T_TOTAL = 4096
BLOCK_T = 512
NUM_BLOCKS = T_TOTAL // BLOCK_T


def _top2_dense_weights(logits):
    idx = jax.lax.broadcasted_iota(jnp.int32, logits.shape, 1)
    m1 = jnp.max(logits, axis=-1, keepdims=True)
    big = jnp.int32(NUM_EXPERTS + 1)
    i1 = jnp.min(jnp.where(logits == m1, idx, big), axis=-1, keepdims=True)
    mask1 = idx == i1
    masked = jnp.where(mask1, -jnp.inf, logits)
    m2 = jnp.max(masked, axis=-1, keepdims=True)
    i2 = jnp.min(jnp.where(masked == m2, idx, big), axis=-1, keepdims=True)
    mask2 = idx == i2
    z = jnp.exp(m2 - m1)
    p1 = 1.0 / (1.0 + z)
    p2 = 1.0 - p1
    return jnp.where(mask1, p1, 0.0) + jnp.where(mask2, p2, 0.0)


def _softmax_colsum(logits):
    m = jnp.max(logits, axis=-1, keepdims=True)
    e = jnp.exp(logits - m)
    p = e / jnp.sum(e, axis=-1, keepdims=True)
    return jnp.sum(p, axis=0, keepdims=True)


def _body(x_ref, rla_ref, rlb_ref, wat_ref, wbt_ref,
          out_ref, auxa_ref, auxb_ref, acca_ref, accb_ref):
    i = pl.program_id(0)
    x = x_ref[...]
    la = rla_ref[...]
    lb = rlb_ref[...]

    wa = _top2_dense_weights(la)
    wb = _top2_dense_weights(lb)

    csa = _softmax_colsum(la)
    csb = _softmax_colsum(lb)

    @pl.when(i == 0)
    def _():
        acca_ref[...] = csa
        accb_ref[...] = csb

    @pl.when(i != 0)
    def _():
        acca_ref[...] += csa
        accb_ref[...] += csb

    mid_all = jnp.dot(x, wat_ref[...], preferred_element_type=jnp.float32)
    mid = wa[:, 0:1] * mid_all[:, 0:R]
    for e in range(1, NUM_EXPERTS):
        mid += wa[:, e:e + 1] * mid_all[:, e * R:(e + 1) * R]

    m_cat = jnp.concatenate(
        [wb[:, e:e + 1] * mid for e in range(NUM_EXPERTS)], axis=1)
    out_ref[...] = jnp.dot(m_cat, wbt_ref[...],
                           preferred_element_type=jnp.float32) * SCALING

    @pl.when(i == NUM_BLOCKS - 1)
    def _():
        inv_t = 1.0 / T_TOTAL
        pa = acca_ref[...] * inv_t
        pb = accb_ref[...] * inv_t
        ma = jnp.sum(pa) / NUM_EXPERTS
        mb = jnp.sum(pb) / NUM_EXPERTS
        va = jnp.sum((pa - ma) ** 2) / (NUM_EXPERTS - 1)
        vb = jnp.sum((pb - mb) ** 2) / (NUM_EXPERTS - 1)
        auxa_ref[...] = (NUM_EXPERTS * va)[None, None]
        auxb_ref[...] = (NUM_EXPERTS * vb)[None, None]


@jax.jit
def _run(flat_x, rla, rlb, wat, wbt):
    out, aux_a, aux_b = pl.pallas_call(
        _body,
        grid=(NUM_BLOCKS,),
        in_specs=[
            pl.BlockSpec((BLOCK_T, IN_FEATURES), lambda i: (i, 0)),
            pl.BlockSpec((BLOCK_T, NUM_EXPERTS), lambda i: (i, 0)),
            pl.BlockSpec((BLOCK_T, NUM_EXPERTS), lambda i: (i, 0)),
            pl.BlockSpec((IN_FEATURES, NUM_EXPERTS * R), lambda i: (0, 0)),
            pl.BlockSpec((NUM_EXPERTS * R, OUT_FEATURES), lambda i: (0, 0)),
        ],
        out_specs=[
            pl.BlockSpec((BLOCK_T, OUT_FEATURES), lambda i: (i, 0)),
            pl.BlockSpec((1, 1), lambda i: (0, 0)),
            pl.BlockSpec((1, 1), lambda i: (0, 0)),
        ],
        out_shape=[
            jax.ShapeDtypeStruct((T_TOTAL, OUT_FEATURES), jnp.float32),
            jax.ShapeDtypeStruct((1, 1), jnp.float32),
            jax.ShapeDtypeStruct((1, 1), jnp.float32),
        ],
        scratch_shapes=[
            pltpu.VMEM((1, NUM_EXPERTS), jnp.float32),
            pltpu.VMEM((1, NUM_EXPERTS), jnp.float32),
        ],
    )(flat_x, rla, rlb, wat, wbt)
    return out, aux_a, aux_b


def kernel(x, router_logits_a, router_logits_b, Wa, Wb):
    batch, seq, _ = x.shape
    flat_x = x.reshape(-1, IN_FEATURES)
    wat = Wa.transpose(2, 0, 1).reshape(IN_FEATURES, NUM_EXPERTS * R)
    wbt = Wb.transpose(0, 2, 1).reshape(NUM_EXPERTS * R, OUT_FEATURES)
    out, aux_a, aux_b = _run(flat_x, router_logits_a, router_logits_b,
                             wat, wbt)
    return (out.reshape(batch, seq, OUT_FEATURES),
            aux_a.reshape(()), aux_b.reshape(()))
